# Initial kernel scaffold; baseline (speedup 1.0000x reference)
import jax
import jax.numpy as jnp
from jax.experimental import pallas as pl


def kernel(x, edge_index, cell_to_spot, num_spots, W1, b1, gn0_w, gn0_b, gn0_ms, Wg1, bg1, gn1_w, gn1_b, gn1_ms, Wg2, bg2, gn2_w, gn2_b, gn2_ms, Wa1, ba1, Wa2, ba2, Wm1, bm1, gn3_w, gn3_b, gn3_ms, Wm2, bm2):
    raise NotImplementedError("write your pallas kernel here")


# speedup vs baseline: 11.9862x; 11.9862x over previous
import functools

import jax
import jax.numpy as jnp
from jax import lax
from jax.experimental import pallas as pl
from jax.experimental.pallas import tpu as pltpu
from jax.experimental.pallas import tpu_sc as plsc

N = 10000
E = 320000
D_IN = 128
PROJ = 64
H1 = 64
H2 = 32
ATTN_H = 32
OUT = 16
NSPOT = 2000

NC = 2
NS = 16
NW = NC * NS
EPW = E // NW
CH = 80
NCHK = EPW // CH
W = 128
NACC = 10112
SROW = NACC // NS

PN = 12288
PPW = PN // NW
PCH = 96
PNCHK = PPW // PCH
ECOL = H2
NSPOTA = 2048
SPOTR = NSPOTA // NS


def _mesh():
    return plsc.VectorSubcoreMesh(core_axis_name="c", subcore_axis_name="s")


def _make_count():
    @functools.partial(
        pl.kernel,
        mesh=_mesh(),
        out_type=jax.ShapeDtypeStruct((NC, NACC, W), jnp.float32),
        scratch_types=[
            pltpu.VMEM((CH,), jnp.int32),
            pltpu.VMEM((CH, W), jnp.float32),
            pltpu.VMEM_SHARED((NACC, W), jnp.float32),
        ],
    )
    def k(dst_hbm, ones_hbm, zeros_hbm, out_hbm, dstf_v, ones_v, acc_sh):
        c = lax.axis_index("c")
        s = lax.axis_index("s")
        wid = s * NC + c
        base = wid * EPW
        pltpu.sync_copy(zeros_hbm.at[pl.ds(s * SROW, SROW)],
                        acc_sh.at[pl.ds(s * SROW, SROW)])
        pltpu.sync_copy(ones_hbm, ones_v)
        plsc.subcore_barrier()

        def body(j, carry):
            off = pl.multiple_of(base + j * CH, 8)
            pltpu.sync_copy(dst_hbm.at[pl.ds(off, CH)], dstf_v)
            pltpu.sync_copy(ones_v, acc_sh.at[dstf_v], add=True)
            return carry

        lax.fori_loop(0, NCHK, body, 0)
        plsc.subcore_barrier()
        pltpu.sync_copy(acc_sh.at[pl.ds(s * SROW, SROW)],
                        out_hbm.at[c, pl.ds(s * SROW, SROW)])

    return k


def _make_edge_scatter():
    @functools.partial(
        pl.kernel,
        mesh=_mesh(),
        out_type=jax.ShapeDtypeStruct((NC, NACC, W), jnp.float32),
        scratch_types=[
            pltpu.VMEM((CH,), jnp.int32),
            pltpu.VMEM((CH,), jnp.int32),
            pltpu.VMEM((CH, W), jnp.float32),
            pltpu.VMEM_SHARED((NACC, W), jnp.float32),
            pltpu.SemaphoreType.DMA,
        ],
    )
    def k(src_hbm, dst_hbm, tab_hbm, zeros_hbm, out_hbm,
          srcf_v, dstf_v, rows_v, acc_sh, sem):
        c = lax.axis_index("c")
        s = lax.axis_index("s")
        wid = s * NC + c
        base = wid * EPW
        pltpu.sync_copy(zeros_hbm.at[pl.ds(s * SROW, SROW)],
                        acc_sh.at[pl.ds(s * SROW, SROW)])
        plsc.subcore_barrier()

        def body(j, carry):
            off = pl.multiple_of(base + j * CH, 8)
            pltpu.sync_copy(src_hbm.at[pl.ds(off, CH)], srcf_v)
            pltpu.sync_copy(dst_hbm.at[pl.ds(off, CH)], dstf_v)
            pltpu.async_copy(tab_hbm.at[srcf_v], rows_v, sem).wait()
            pltpu.sync_copy(rows_v, acc_sh.at[dstf_v], add=True)
            return carry

        lax.fori_loop(0, NCHK, body, 0)
        plsc.subcore_barrier()
        pltpu.sync_copy(acc_sh.at[pl.ds(s * SROW, SROW)],
                        out_hbm.at[c, pl.ds(s * SROW, SROW)])

    return k


def _make_pool():
    @functools.partial(
        pl.kernel,
        mesh=_mesh(),
        out_type=jax.ShapeDtypeStruct((NC, NSPOTA, W), jnp.float32),
        scratch_types=[
            pltpu.VMEM((PCH,), jnp.int32),
            pltpu.VMEM((PPW, W), jnp.float32),
            pltpu.VMEM_SHARED((NSPOTA, W), jnp.float32),
        ],
    )
    def k(tab_hbm, sidx_hbm, zeros_hbm, out_hbm, idxf_v, rows_v, acc_sh):
        c = lax.axis_index("c")
        s = lax.axis_index("s")
        wid = s * NC + c
        pltpu.sync_copy(zeros_hbm.at[pl.ds(s * SPOTR, SPOTR)],
                        acc_sh.at[pl.ds(s * SPOTR, SPOTR)])
        pltpu.sync_copy(tab_hbm.at[pl.ds(wid * PPW, PPW)], rows_v)
        plsc.subcore_barrier()
        for j in range(PNCHK):
            pltpu.sync_copy(sidx_hbm.at[pl.ds(wid * PPW + j * PCH, PCH)], idxf_v)
            pltpu.sync_copy(rows_v.at[pl.ds(j * PCH, PCH)],
                            acc_sh.at[idxf_v], add=True)
        plsc.subcore_barrier()
        pltpu.sync_copy(acc_sh.at[pl.ds(s * SPOTR, SPOTR)],
                        out_hbm.at[c, pl.ds(s * SPOTR, SPOTR)])

    return k


_count = _make_count()
_scat = _make_edge_scatter()
_pool = _make_pool()


def _gnorm(h, w, b, ms):
    mean = jnp.mean(h, axis=0, keepdims=True)
    out = h - ms * mean
    var = jnp.mean(out * out, axis=0, keepdims=True)
    return w * out / jnp.sqrt(var + 1e-5) + b


def _dis_of(deg_ref):
    deg = deg_ref[0, :N, 0] + deg_ref[1, :N, 0] + 1.0
    return lax.rsqrt(deg)


def _pad_table(t):
    n, d = t.shape
    return jnp.pad(t, ((0, NACC - n), (0, W - d)))


def _tc1_body(x_ref, w1_ref, b1_ref, gw_ref, gb_ref, gms_ref, wg1_ref,
              deg_ref, g1p_ref):
    h = jnp.dot(x_ref[...], w1_ref[...], preferred_element_type=jnp.float32)
    h = _gnorm(h + b1_ref[...], gw_ref[...], gb_ref[...], gms_ref[...])
    h = jnp.maximum(h, 0.0)
    dis = _dis_of(deg_ref)
    g1 = jnp.dot(h, wg1_ref[...], preferred_element_type=jnp.float32)
    g1p_ref[...] = _pad_table(g1 * dis[:, None])


def _tc2_body(g1p_ref, acc_ref, deg_ref, bg1_ref, gw_ref, gb_ref, gms_ref,
              wg2_ref, g2p_ref):
    dis = _dis_of(deg_ref)
    agg = acc_ref[0, :N, :PROJ] + acc_ref[1, :N, :PROJ] + g1p_ref[:N, :PROJ]
    t = dis[:, None] * agg + bg1_ref[...]
    h = jnp.maximum(_gnorm(t, gw_ref[...], gb_ref[...], gms_ref[...]), 0.0)
    g2 = jnp.dot(h, wg2_ref[...], preferred_element_type=jnp.float32)
    g2p_ref[...] = _pad_table(g2 * dis[:, None])


def _tc3_body(g2p_ref, acc_ref, deg_ref, bg2_ref, gw_ref, gb_ref, gms_ref,
              wa1_ref, ba1_ref, wa2_ref, ba2_ref, t_ref):
    dis = _dis_of(deg_ref)
    agg = acc_ref[0, :N, :H2] + acc_ref[1, :N, :H2] + g2p_ref[:N, :H2]
    t = dis[:, None] * agg + bg2_ref[...]
    h2 = jnp.maximum(_gnorm(t, gw_ref[...], gb_ref[...], gms_ref[...]), 0.0)
    a = jnp.maximum(
        jnp.dot(h2, wa1_ref[...], preferred_element_type=jnp.float32)
        + ba1_ref[...], 0.0)
    sc = jnp.dot(a, wa2_ref[...], preferred_element_type=jnp.float32)[:, 0]
    sc = sc + ba2_ref[0, 0]
    e = jnp.exp(sc - jnp.max(sc))
    body = jnp.concatenate(
        [h2 * e[:, None], e[:, None], jnp.zeros((N, W - H2 - 1), jnp.float32)],
        axis=1)
    t_ref[...] = jnp.concatenate(
        [body, jnp.zeros((PN - N, W), jnp.float32)], axis=0)


def _tc4_body(acc_ref, wm1_ref, bm1_ref, gw_ref, gb_ref, gms_ref,
              wm2_ref, bm2_ref, out_ref):
    num = acc_ref[0, :NSPOT, :H2] + acc_ref[1, :NSPOT, :H2]
    den = acc_ref[0, :NSPOT, ECOL] + acc_ref[1, :NSPOT, ECOL]
    spot = num / (den + 1e-16)[:, None]
    z = jnp.dot(spot, wm1_ref[...], preferred_element_type=jnp.float32)
    z = jnp.maximum(_gnorm(z + bm1_ref[...], gw_ref[...], gb_ref[...],
                           gms_ref[...]), 0.0)
    out_ref[...] = jnp.dot(z, wm2_ref[...],
                           preferred_element_type=jnp.float32) + bm2_ref[...]


def _tc(body, out_shape, *args):
    return pl.pallas_call(body, out_shape=out_shape)(*args)


def kernel(x, edge_index, cell_to_spot, num_spots,
           W1, b1, gn0_w, gn0_b, gn0_ms,
           Wg1, bg1, gn1_w, gn1_b, gn1_ms,
           Wg2, bg2, gn2_w, gn2_b, gn2_ms,
           Wa1, ba1, Wa2, ba2,
           Wm1, bm1, gn3_w, gn3_b, gn3_ms,
           Wm2, bm2):
    del num_spots
    r = lambda v: v.reshape(1, -1)
    ei = edge_index.astype(jnp.int32)
    src1 = ei[0]
    dst1 = ei[1]
    pad_idx = jnp.arange(PN - N, dtype=jnp.int32) % NSPOT
    cs1 = jnp.concatenate([cell_to_spot.astype(jnp.int32), pad_idx])

    zeros_acc = jnp.zeros((NACC, W), jnp.float32)
    degacc = _count(dst1, jnp.ones((CH, W), jnp.float32), zeros_acc)

    g1p = _tc(_tc1_body, jax.ShapeDtypeStruct((NACC, W), jnp.float32),
              x, W1, r(b1), r(gn0_w), r(gn0_b), r(gn0_ms), Wg1, degacc)
    acc1 = _scat(src1, dst1, g1p, zeros_acc)

    g2p = _tc(_tc2_body, jax.ShapeDtypeStruct((NACC, W), jnp.float32),
              g1p, acc1, degacc, r(bg1), r(gn1_w), r(gn1_b), r(gn1_ms), Wg2)
    acc2 = _scat(src1, dst1, g2p, zeros_acc)

    tbl = _tc(_tc3_body, jax.ShapeDtypeStruct((PN, W), jnp.float32),
              g2p, acc2, degacc, r(bg2), r(gn2_w), r(gn2_b), r(gn2_ms),
              Wa1, r(ba1), Wa2, r(ba2))
    spotacc = _pool(tbl, cs1, jnp.zeros((NSPOTA, W), jnp.float32))

    return _tc(_tc4_body, jax.ShapeDtypeStruct((NSPOT, OUT), jnp.float32),
               spotacc, Wm1, r(bm1), r(gn3_w), r(gn3_b), r(gn3_ms),
               Wm2, r(bm2))


# speedup vs baseline: 17.1579x; 1.4315x over previous
import functools

import jax
import jax.numpy as jnp
from jax import lax
from jax.experimental import pallas as pl
from jax.experimental.pallas import tpu as pltpu
from jax.experimental.pallas import tpu_sc as plsc

N = 10000
E = 320000
D_IN = 128
PROJ = 64
H1 = 64
H2 = 32
ATTN_H = 32
OUT = 16
NSPOT = 2000

NC = 2
NS = 16
NW = NC * NS
EPW = E // NW
CH = 80
NCHK = EPW // CH
W = 128
NACC = 10112
SROW = NACC // NS

PN = 12288
PPW = PN // NW
PCH = 96
PNCHK = PPW // PCH
ECOL = H2
NSPOTA = 2048
SPOTR = NSPOTA // NS


def _mesh():
    return plsc.VectorSubcoreMesh(core_axis_name="c", subcore_axis_name="s")


def _make_count():
    @functools.partial(
        pl.kernel,
        mesh=_mesh(),
        out_type=jax.ShapeDtypeStruct((NC, NACC, W), jnp.float32),
        scratch_types=[
            pltpu.VMEM((CH,), jnp.int32),
            pltpu.VMEM((CH,), jnp.int32),
            pltpu.VMEM((CH, W), jnp.float32),
            pltpu.VMEM_SHARED((NACC, W), jnp.float32),
        ],
    )
    def k(dst_hbm, ones_hbm, zeros_hbm, out_hbm, dstf_v, dstf2_v, ones_v, acc_sh):
        c = lax.axis_index("c")
        s = lax.axis_index("s")
        wid = s * NC + c
        base = wid * EPW
        pltpu.sync_copy(zeros_hbm.at[pl.ds(s * SROW, SROW)],
                        acc_sh.at[pl.ds(s * SROW, SROW)])
        pltpu.sync_copy(ones_hbm, ones_v)
        plsc.subcore_barrier()

        def off(j):
            return pl.multiple_of(base + j * CH, 8)

        pltpu.sync_copy(dst_hbm.at[pl.ds(off(0), CH)], dstf_v)

        def body(i, carry):
            j0 = 2 * i
            pltpu.sync_copy(dst_hbm.at[pl.ds(off(j0 + 1), CH)], dstf2_v)
            pltpu.sync_copy(ones_v, acc_sh.at[dstf_v], add=True)
            pltpu.sync_copy(dst_hbm.at[pl.ds(off(j0 + 2), CH)], dstf_v)
            pltpu.sync_copy(ones_v, acc_sh.at[dstf2_v], add=True)
            return carry

        lax.fori_loop(0, (NCHK - 1) // 2, body, 0)
        pltpu.sync_copy(ones_v, acc_sh.at[dstf_v], add=True)
        plsc.subcore_barrier()
        pltpu.sync_copy(acc_sh.at[pl.ds(s * SROW, SROW)],
                        out_hbm.at[c, pl.ds(s * SROW, SROW)])

    return k


def _make_edge_scatter():
    @functools.partial(
        pl.kernel,
        mesh=_mesh(),
        out_type=jax.ShapeDtypeStruct((NC, NACC, W), jnp.float32),
        scratch_types=[
            pltpu.VMEM((CH,), jnp.int32),
            pltpu.VMEM((CH,), jnp.int32),
            pltpu.VMEM((CH,), jnp.int32),
            pltpu.VMEM((CH,), jnp.int32),
            pltpu.VMEM((CH, W), jnp.float32),
            pltpu.VMEM((CH, W), jnp.float32),
            pltpu.VMEM_SHARED((NACC, W), jnp.float32),
            pltpu.SemaphoreType.DMA,
            pltpu.SemaphoreType.DMA,
        ],
    )
    def k(src_hbm, dst_hbm, tab_hbm, zeros_hbm, out_hbm,
          srca_v, dsta_v, srcb_v, dstb_v, rowsa_v, rowsb_v, acc_sh,
          gsema, gsemb):
        c = lax.axis_index("c")
        s = lax.axis_index("s")
        wid = s * NC + c
        base = wid * EPW
        pltpu.sync_copy(zeros_hbm.at[pl.ds(s * SROW, SROW)],
                        acc_sh.at[pl.ds(s * SROW, SROW)])
        plsc.subcore_barrier()

        def off(j):
            return pl.multiple_of(base + j * CH, 8)

        pltpu.sync_copy(src_hbm.at[pl.ds(off(0), CH)], srca_v)
        pltpu.sync_copy(dst_hbm.at[pl.ds(off(0), CH)], dsta_v)
        pltpu.async_copy(tab_hbm.at[srca_v], rowsa_v, gsema)

        def body(i, carry):
            j0 = 2 * i
            pltpu.sync_copy(src_hbm.at[pl.ds(off(j0 + 1), CH)], srcb_v)
            pltpu.sync_copy(dst_hbm.at[pl.ds(off(j0 + 1), CH)], dstb_v)
            pltpu.async_copy(tab_hbm.at[srcb_v], rowsb_v, gsemb)
            pltpu.make_async_copy(tab_hbm.at[srca_v], rowsa_v, gsema).wait()
            pltpu.sync_copy(rowsa_v, acc_sh.at[dsta_v], add=True)
            pltpu.sync_copy(src_hbm.at[pl.ds(off(j0 + 2), CH)], srca_v)
            pltpu.sync_copy(dst_hbm.at[pl.ds(off(j0 + 2), CH)], dsta_v)
            pltpu.async_copy(tab_hbm.at[srca_v], rowsa_v, gsema)
            pltpu.make_async_copy(tab_hbm.at[srcb_v], rowsb_v, gsemb).wait()
            pltpu.sync_copy(rowsb_v, acc_sh.at[dstb_v], add=True)
            return carry

        lax.fori_loop(0, (NCHK - 1) // 2, body, 0)
        pltpu.make_async_copy(tab_hbm.at[srca_v], rowsa_v, gsema).wait()
        pltpu.sync_copy(rowsa_v, acc_sh.at[dsta_v], add=True)
        plsc.subcore_barrier()
        pltpu.sync_copy(acc_sh.at[pl.ds(s * SROW, SROW)],
                        out_hbm.at[c, pl.ds(s * SROW, SROW)])

    return k


def _make_pool():
    @functools.partial(
        pl.kernel,
        mesh=_mesh(),
        out_type=jax.ShapeDtypeStruct((NC, NSPOTA, W), jnp.float32),
        scratch_types=[
            pltpu.VMEM((PCH,), jnp.int32),
            pltpu.VMEM((PPW, W), jnp.float32),
            pltpu.VMEM_SHARED((NSPOTA, W), jnp.float32),
        ],
    )
    def k(tab_hbm, sidx_hbm, zeros_hbm, out_hbm, idxf_v, rows_v, acc_sh):
        c = lax.axis_index("c")
        s = lax.axis_index("s")
        wid = s * NC + c
        pltpu.sync_copy(zeros_hbm.at[pl.ds(s * SPOTR, SPOTR)],
                        acc_sh.at[pl.ds(s * SPOTR, SPOTR)])
        pltpu.sync_copy(tab_hbm.at[pl.ds(wid * PPW, PPW)], rows_v)
        plsc.subcore_barrier()
        for j in range(PNCHK):
            pltpu.sync_copy(sidx_hbm.at[pl.ds(wid * PPW + j * PCH, PCH)], idxf_v)
            pltpu.sync_copy(rows_v.at[pl.ds(j * PCH, PCH)],
                            acc_sh.at[idxf_v], add=True)
        plsc.subcore_barrier()
        pltpu.sync_copy(acc_sh.at[pl.ds(s * SPOTR, SPOTR)],
                        out_hbm.at[c, pl.ds(s * SPOTR, SPOTR)])

    return k


_count = _make_count()
_scat = _make_edge_scatter()
_pool = _make_pool()


def _gnorm(h, w, b, ms):
    mean = jnp.mean(h, axis=0, keepdims=True)
    out = h - ms * mean
    var = jnp.mean(out * out, axis=0, keepdims=True)
    return w * out / jnp.sqrt(var + 1e-5) + b


def _dis_of(deg_ref):
    deg = deg_ref[0, :N, 0] + deg_ref[1, :N, 0] + 1.0
    return lax.rsqrt(deg)


def _pad_table(t):
    n, d = t.shape
    return jnp.pad(t, ((0, NACC - n), (0, W - d)))


def _tc1_body(x_ref, w1_ref, b1_ref, gw_ref, gb_ref, gms_ref, wg1_ref,
              deg_ref, g1p_ref):
    h = jnp.dot(x_ref[...], w1_ref[...], preferred_element_type=jnp.float32)
    h = _gnorm(h + b1_ref[...], gw_ref[...], gb_ref[...], gms_ref[...])
    h = jnp.maximum(h, 0.0)
    dis = _dis_of(deg_ref)
    g1 = jnp.dot(h, wg1_ref[...], preferred_element_type=jnp.float32)
    g1p_ref[...] = _pad_table(g1 * dis[:, None])


def _tc2_body(g1p_ref, acc_ref, deg_ref, bg1_ref, gw_ref, gb_ref, gms_ref,
              wg2_ref, g2p_ref):
    dis = _dis_of(deg_ref)
    agg = acc_ref[0, :N, :PROJ] + acc_ref[1, :N, :PROJ] + g1p_ref[:N, :PROJ]
    t = dis[:, None] * agg + bg1_ref[...]
    h = jnp.maximum(_gnorm(t, gw_ref[...], gb_ref[...], gms_ref[...]), 0.0)
    g2 = jnp.dot(h, wg2_ref[...], preferred_element_type=jnp.float32)
    g2p_ref[...] = _pad_table(g2 * dis[:, None])


def _tc3_body(g2p_ref, acc_ref, deg_ref, bg2_ref, gw_ref, gb_ref, gms_ref,
              wa1_ref, ba1_ref, wa2_ref, ba2_ref, t_ref):
    dis = _dis_of(deg_ref)
    agg = acc_ref[0, :N, :H2] + acc_ref[1, :N, :H2] + g2p_ref[:N, :H2]
    t = dis[:, None] * agg + bg2_ref[...]
    h2 = jnp.maximum(_gnorm(t, gw_ref[...], gb_ref[...], gms_ref[...]), 0.0)
    a = jnp.maximum(
        jnp.dot(h2, wa1_ref[...], preferred_element_type=jnp.float32)
        + ba1_ref[...], 0.0)
    sc = jnp.dot(a, wa2_ref[...], preferred_element_type=jnp.float32)[:, 0]
    sc = sc + ba2_ref[0, 0]
    e = jnp.exp(sc - jnp.max(sc))
    body = jnp.concatenate(
        [h2 * e[:, None], e[:, None], jnp.zeros((N, W - H2 - 1), jnp.float32)],
        axis=1)
    t_ref[...] = jnp.concatenate(
        [body, jnp.zeros((PN - N, W), jnp.float32)], axis=0)


def _tc4_body(acc_ref, wm1_ref, bm1_ref, gw_ref, gb_ref, gms_ref,
              wm2_ref, bm2_ref, out_ref):
    num = acc_ref[0, :NSPOT, :H2] + acc_ref[1, :NSPOT, :H2]
    den = acc_ref[0, :NSPOT, ECOL] + acc_ref[1, :NSPOT, ECOL]
    spot = num / (den + 1e-16)[:, None]
    z = jnp.dot(spot, wm1_ref[...], preferred_element_type=jnp.float32)
    z = jnp.maximum(_gnorm(z + bm1_ref[...], gw_ref[...], gb_ref[...],
                           gms_ref[...]), 0.0)
    out_ref[...] = jnp.dot(z, wm2_ref[...],
                           preferred_element_type=jnp.float32) + bm2_ref[...]


def _tc(body, out_shape, *args):
    return pl.pallas_call(body, out_shape=out_shape)(*args)


def kernel(x, edge_index, cell_to_spot, num_spots,
           W1, b1, gn0_w, gn0_b, gn0_ms,
           Wg1, bg1, gn1_w, gn1_b, gn1_ms,
           Wg2, bg2, gn2_w, gn2_b, gn2_ms,
           Wa1, ba1, Wa2, ba2,
           Wm1, bm1, gn3_w, gn3_b, gn3_ms,
           Wm2, bm2):
    del num_spots
    r = lambda v: v.reshape(1, -1)
    ei = edge_index.astype(jnp.int32)
    src1 = ei[0]
    dst1 = ei[1]
    pad_idx = jnp.arange(PN - N, dtype=jnp.int32) % NSPOT
    cs1 = jnp.concatenate([cell_to_spot.astype(jnp.int32), pad_idx])

    zeros_acc = jnp.zeros((NACC, W), jnp.float32)
    degacc = _count(dst1, jnp.ones((CH, W), jnp.float32), zeros_acc)

    g1p = _tc(_tc1_body, jax.ShapeDtypeStruct((NACC, W), jnp.float32),
              x, W1, r(b1), r(gn0_w), r(gn0_b), r(gn0_ms), Wg1, degacc)
    acc1 = _scat(src1, dst1, g1p, zeros_acc)

    g2p = _tc(_tc2_body, jax.ShapeDtypeStruct((NACC, W), jnp.float32),
              g1p, acc1, degacc, r(bg1), r(gn1_w), r(gn1_b), r(gn1_ms), Wg2)
    acc2 = _scat(src1, dst1, g2p, zeros_acc)

    tbl = _tc(_tc3_body, jax.ShapeDtypeStruct((PN, W), jnp.float32),
              g2p, acc2, degacc, r(bg2), r(gn2_w), r(gn2_b), r(gn2_ms),
              Wa1, r(ba1), Wa2, r(ba2))
    spotacc = _pool(tbl, cs1, jnp.zeros((NSPOTA, W), jnp.float32))

    return _tc(_tc4_body, jax.ShapeDtypeStruct((NSPOT, OUT), jnp.float32),
               spotacc, Wm1, r(bm1), r(gn3_w), r(gn3_b), r(gn3_ms),
               Wm2, r(bm2))


# speedup vs baseline: 24.7403x; 1.4419x over previous
import functools

import jax
import jax.numpy as jnp
from jax import lax
from jax.experimental import pallas as pl
from jax.experimental.pallas import tpu as pltpu
from jax.experimental.pallas import tpu_sc as plsc

N = 10000
E = 320000
D_IN = 128
PROJ = 64
H1 = 64
H2 = 32
ATTN_H = 32
OUT = 16
NSPOT = 2000

NC = 2
NS = 16
NW = NC * NS
EPW = E // NW
CH = 80
NCHK = EPW // CH
W = 128
NACC = 10112
SROW = NACC // NS

PN = 12288
PPW = PN // NW
PCH = 96
PNCHK = PPW // PCH
ECOL = H2
NSPOTA = 2048
SPOTR = NSPOTA // NS


def _mesh():
    return plsc.VectorSubcoreMesh(core_axis_name="c", subcore_axis_name="s")


def _make_count():
    @functools.partial(
        pl.kernel,
        mesh=_mesh(),
        out_type=jax.ShapeDtypeStruct((NC, NACC, W), jnp.float32),
        scratch_types=[
            pltpu.VMEM((EPW,), jnp.int32),
            pltpu.VMEM((CH, W), jnp.float32),
            pltpu.VMEM_SHARED((NACC, W), jnp.float32),
        ],
    )
    def k(dst_hbm, ones_hbm, zeros_hbm, out_hbm, dst_v, ones_v, acc_sh):
        c = lax.axis_index("c")
        s = lax.axis_index("s")
        wid = s * NC + c
        pltpu.sync_copy(zeros_hbm.at[pl.ds(s * SROW, SROW)],
                        acc_sh.at[pl.ds(s * SROW, SROW)])
        pltpu.sync_copy(dst_hbm.at[pl.ds(wid * EPW, EPW)], dst_v)
        pltpu.sync_copy(ones_hbm, ones_v)
        plsc.subcore_barrier()

        def body(j, carry):
            o = pl.multiple_of(j * CH, 8)
            pltpu.sync_copy(ones_v, acc_sh.at[dst_v.at[pl.ds(o, CH)]],
                            add=True)
            return carry

        lax.fori_loop(0, NCHK, body, 0)
        plsc.subcore_barrier()
        pltpu.sync_copy(acc_sh.at[pl.ds(s * SROW, SROW)],
                        out_hbm.at[c, pl.ds(s * SROW, SROW)])

    return k


def _make_edge_scatter():
    @functools.partial(
        pl.kernel,
        mesh=_mesh(),
        out_type=jax.ShapeDtypeStruct((NC, NACC, W), jnp.float32),
        scratch_types=[
            pltpu.VMEM((EPW,), jnp.int32),
            pltpu.VMEM((EPW,), jnp.int32),
            pltpu.VMEM((CH, W), jnp.float32),
            pltpu.VMEM((CH, W), jnp.float32),
            pltpu.VMEM_SHARED((NACC, W), jnp.float32),
            pltpu.SemaphoreType.DMA,
            pltpu.SemaphoreType.DMA,
        ],
    )
    def k(src_hbm, dst_hbm, tab_hbm, zeros_hbm, out_hbm,
          src_v, dst_v, rowsa_v, rowsb_v, acc_sh, gsema, gsemb):
        c = lax.axis_index("c")
        s = lax.axis_index("s")
        wid = s * NC + c
        pltpu.sync_copy(zeros_hbm.at[pl.ds(s * SROW, SROW)],
                        acc_sh.at[pl.ds(s * SROW, SROW)])
        pltpu.sync_copy(src_hbm.at[pl.ds(wid * EPW, EPW)], src_v)
        pltpu.sync_copy(dst_hbm.at[pl.ds(wid * EPW, EPW)], dst_v)
        plsc.subcore_barrier()

        def sl(ref, j):
            return ref.at[pl.ds(pl.multiple_of(j * CH, 8), CH)]

        pltpu.async_copy(tab_hbm.at[sl(src_v, 0)], rowsa_v, gsema)

        def body(i, carry):
            j0 = 2 * i
            pltpu.async_copy(tab_hbm.at[sl(src_v, j0 + 1)], rowsb_v, gsemb)
            pltpu.make_async_copy(tab_hbm.at[sl(src_v, j0)], rowsa_v,
                                  gsema).wait()
            pltpu.sync_copy(rowsa_v, acc_sh.at[sl(dst_v, j0)], add=True)
            pltpu.async_copy(tab_hbm.at[sl(src_v, j0 + 2)], rowsa_v, gsema)
            pltpu.make_async_copy(tab_hbm.at[sl(src_v, j0 + 1)], rowsb_v,
                                  gsemb).wait()
            pltpu.sync_copy(rowsb_v, acc_sh.at[sl(dst_v, j0 + 1)], add=True)
            return carry

        lax.fori_loop(0, (NCHK - 1) // 2, body, 0)
        pltpu.make_async_copy(tab_hbm.at[sl(src_v, NCHK - 1)], rowsa_v,
                              gsema).wait()
        pltpu.sync_copy(rowsa_v, acc_sh.at[sl(dst_v, NCHK - 1)], add=True)
        plsc.subcore_barrier()
        pltpu.sync_copy(acc_sh.at[pl.ds(s * SROW, SROW)],
                        out_hbm.at[c, pl.ds(s * SROW, SROW)])

    return k


def _make_pool():
    @functools.partial(
        pl.kernel,
        mesh=_mesh(),
        out_type=jax.ShapeDtypeStruct((NC, NSPOTA, W), jnp.float32),
        scratch_types=[
            pltpu.VMEM((PCH,), jnp.int32),
            pltpu.VMEM((PPW, W), jnp.float32),
            pltpu.VMEM_SHARED((NSPOTA, W), jnp.float32),
        ],
    )
    def k(tab_hbm, sidx_hbm, zeros_hbm, out_hbm, idxf_v, rows_v, acc_sh):
        c = lax.axis_index("c")
        s = lax.axis_index("s")
        wid = s * NC + c
        pltpu.sync_copy(zeros_hbm.at[pl.ds(s * SPOTR, SPOTR)],
                        acc_sh.at[pl.ds(s * SPOTR, SPOTR)])
        pltpu.sync_copy(tab_hbm.at[pl.ds(wid * PPW, PPW)], rows_v)
        plsc.subcore_barrier()
        for j in range(PNCHK):
            pltpu.sync_copy(sidx_hbm.at[pl.ds(wid * PPW + j * PCH, PCH)], idxf_v)
            pltpu.sync_copy(rows_v.at[pl.ds(j * PCH, PCH)],
                            acc_sh.at[idxf_v], add=True)
        plsc.subcore_barrier()
        pltpu.sync_copy(acc_sh.at[pl.ds(s * SPOTR, SPOTR)],
                        out_hbm.at[c, pl.ds(s * SPOTR, SPOTR)])

    return k


_count = _make_count()
_scat = _make_edge_scatter()
_pool = _make_pool()


def _gnorm(h, w, b, ms):
    mean = jnp.mean(h, axis=0, keepdims=True)
    out = h - ms * mean
    var = jnp.mean(out * out, axis=0, keepdims=True)
    return w * out / jnp.sqrt(var + 1e-5) + b


def _dis_of(deg_ref):
    deg = deg_ref[0, :N, 0] + deg_ref[1, :N, 0] + 1.0
    return lax.rsqrt(deg)


def _pad_table(t):
    n, d = t.shape
    return jnp.pad(t, ((0, NACC - n), (0, W - d)))


def _tc1_body(x_ref, w1_ref, b1_ref, gw_ref, gb_ref, gms_ref, wg1_ref,
              deg_ref, g1p_ref):
    h = jnp.dot(x_ref[...], w1_ref[...], preferred_element_type=jnp.float32)
    h = _gnorm(h + b1_ref[...], gw_ref[...], gb_ref[...], gms_ref[...])
    h = jnp.maximum(h, 0.0)
    dis = _dis_of(deg_ref)
    g1 = jnp.dot(h, wg1_ref[...], preferred_element_type=jnp.float32)
    g1p_ref[...] = _pad_table(g1 * dis[:, None])


def _tc2_body(g1p_ref, acc_ref, deg_ref, bg1_ref, gw_ref, gb_ref, gms_ref,
              wg2_ref, g2p_ref):
    dis = _dis_of(deg_ref)
    agg = acc_ref[0, :N, :PROJ] + acc_ref[1, :N, :PROJ] + g1p_ref[:N, :PROJ]
    t = dis[:, None] * agg + bg1_ref[...]
    h = jnp.maximum(_gnorm(t, gw_ref[...], gb_ref[...], gms_ref[...]), 0.0)
    g2 = jnp.dot(h, wg2_ref[...], preferred_element_type=jnp.float32)
    g2p_ref[...] = _pad_table(g2 * dis[:, None])


def _tc3_body(g2p_ref, acc_ref, deg_ref, bg2_ref, gw_ref, gb_ref, gms_ref,
              wa1_ref, ba1_ref, wa2_ref, ba2_ref, t_ref):
    dis = _dis_of(deg_ref)
    agg = acc_ref[0, :N, :H2] + acc_ref[1, :N, :H2] + g2p_ref[:N, :H2]
    t = dis[:, None] * agg + bg2_ref[...]
    h2 = jnp.maximum(_gnorm(t, gw_ref[...], gb_ref[...], gms_ref[...]), 0.0)
    a = jnp.maximum(
        jnp.dot(h2, wa1_ref[...], preferred_element_type=jnp.float32)
        + ba1_ref[...], 0.0)
    sc = jnp.dot(a, wa2_ref[...], preferred_element_type=jnp.float32)[:, 0]
    sc = sc + ba2_ref[0, 0]
    e = jnp.exp(sc - jnp.max(sc))
    body = jnp.concatenate(
        [h2 * e[:, None], e[:, None], jnp.zeros((N, W - H2 - 1), jnp.float32)],
        axis=1)
    t_ref[...] = jnp.concatenate(
        [body, jnp.zeros((PN - N, W), jnp.float32)], axis=0)


def _tc4_body(acc_ref, wm1_ref, bm1_ref, gw_ref, gb_ref, gms_ref,
              wm2_ref, bm2_ref, out_ref):
    num = acc_ref[0, :NSPOT, :H2] + acc_ref[1, :NSPOT, :H2]
    den = acc_ref[0, :NSPOT, ECOL] + acc_ref[1, :NSPOT, ECOL]
    spot = num / (den + 1e-16)[:, None]
    z = jnp.dot(spot, wm1_ref[...], preferred_element_type=jnp.float32)
    z = jnp.maximum(_gnorm(z + bm1_ref[...], gw_ref[...], gb_ref[...],
                           gms_ref[...]), 0.0)
    out_ref[...] = jnp.dot(z, wm2_ref[...],
                           preferred_element_type=jnp.float32) + bm2_ref[...]


def _tc(body, out_shape, *args):
    return pl.pallas_call(body, out_shape=out_shape)(*args)


def kernel(x, edge_index, cell_to_spot, num_spots,
           W1, b1, gn0_w, gn0_b, gn0_ms,
           Wg1, bg1, gn1_w, gn1_b, gn1_ms,
           Wg2, bg2, gn2_w, gn2_b, gn2_ms,
           Wa1, ba1, Wa2, ba2,
           Wm1, bm1, gn3_w, gn3_b, gn3_ms,
           Wm2, bm2):
    del num_spots
    r = lambda v: v.reshape(1, -1)
    ei = edge_index.astype(jnp.int32)
    src1 = ei[0]
    dst1 = ei[1]
    pad_idx = jnp.arange(PN - N, dtype=jnp.int32) % NSPOT
    cs1 = jnp.concatenate([cell_to_spot.astype(jnp.int32), pad_idx])

    zeros_acc = jnp.zeros((NACC, W), jnp.float32)
    degacc = _count(dst1, jnp.ones((CH, W), jnp.float32), zeros_acc)

    g1p = _tc(_tc1_body, jax.ShapeDtypeStruct((NACC, W), jnp.float32),
              x, W1, r(b1), r(gn0_w), r(gn0_b), r(gn0_ms), Wg1, degacc)
    acc1 = _scat(src1, dst1, g1p, zeros_acc)

    g2p = _tc(_tc2_body, jax.ShapeDtypeStruct((NACC, W), jnp.float32),
              g1p, acc1, degacc, r(bg1), r(gn1_w), r(gn1_b), r(gn1_ms), Wg2)
    acc2 = _scat(src1, dst1, g2p, zeros_acc)

    tbl = _tc(_tc3_body, jax.ShapeDtypeStruct((PN, W), jnp.float32),
              g2p, acc2, degacc, r(bg2), r(gn2_w), r(gn2_b), r(gn2_ms),
              Wa1, r(ba1), Wa2, r(ba2))
    spotacc = _pool(tbl, cs1, jnp.zeros((NSPOTA, W), jnp.float32))

    return _tc(_tc4_body, jax.ShapeDtypeStruct((NSPOT, OUT), jnp.float32),
               spotacc, Wm1, r(bm1), r(gn3_w), r(gn3_b), r(gn3_ms),
               Wm2, r(bm2))
